# baseline (speedup 1.0000x reference)
import functools
import math

import jax
import jax.numpy as jnp
from jax import lax
from jax.experimental import pallas as pl
from jax.experimental.pallas import tpu as pltpu
from jax.experimental.pallas import tpu_sc as plsc

D_MODEL = 64
SCALE = 8.0
LANES = 16

NUM_CORES = 2
NUM_SUBCORES = 16
NUM_WORKERS = NUM_CORES * NUM_SUBCORES

IDX_W = 128
SUBS = 4
CHUNK = IDX_W * SUBS


def _emb_body(idx_hbm, table_hbm, out_hbm, idx_v, rows_v, sem, *, n_chunks,
              per_w):
  wid = lax.axis_index("s") * NUM_CORES + lax.axis_index("c")
  base = wid * per_w

  def chunk_body(g, carry):
    off = base + g * CHUNK
    pltpu.sync_copy(idx_hbm.at[pl.ds(off, CHUNK)], idx_v)
    descs = [
        pltpu.async_copy(table_hbm.at[idx_v.at[pl.ds(j * IDX_W, IDX_W)]],
                         rows_v.at[pl.ds(j * IDX_W, IDX_W)], sem)
        for j in range(SUBS)
    ]
    for d in descs:
      d.wait()
    def scale_body(r, c):
      for l in range(D_MODEL // LANES):
        sl = pl.ds(l * LANES, LANES)
        rows_v[r, sl] = rows_v[r, sl] * SCALE
      return c
    lax.fori_loop(0, CHUNK, scale_body, 0)
    pltpu.sync_copy(rows_v, out_hbm.at[pl.ds(off, CHUNK)])
    return carry

  lax.fori_loop(0, n_chunks, chunk_body, 0)


def kernel(x, table):
  b0, b1 = x.shape
  b = b0 * b1
  assert b % (NUM_WORKERS * CHUNK) == 0
  per_w = b // NUM_WORKERS
  n_chunks = per_w // CHUNK

  idx = x.reshape(b).astype(jnp.int32)

  mesh = plsc.VectorSubcoreMesh(core_axis_name="c", subcore_axis_name="s")
  emb = pl.kernel(
      functools.partial(_emb_body, n_chunks=n_chunks, per_w=per_w),
      out_type=jax.ShapeDtypeStruct((b, D_MODEL), jnp.float32),
      mesh=mesh,
      compiler_params=pltpu.CompilerParams(use_tc_tiling_on_sc=False),
      scratch_types=[
          pltpu.VMEM((CHUNK,), jnp.int32),
          pltpu.VMEM((CHUNK, D_MODEL), jnp.float32),
          pltpu.SemaphoreType.DMA,
      ],
  )
  out = emb(idx, table)
  return out.reshape(b0, b1, D_MODEL)


# speedup vs baseline: 1.1337x; 1.1337x over previous
import functools
import math

import jax
import jax.numpy as jnp
from jax import lax
from jax.experimental import pallas as pl
from jax.experimental.pallas import tpu as pltpu
from jax.experimental.pallas import tpu_sc as plsc

D_MODEL = 64
SCALE = 8.0
LANES = 16

NUM_CORES = 2
NUM_SUBCORES = 16
NUM_WORKERS = NUM_CORES * NUM_SUBCORES

IDX_W = 128
SUBS = 4
CHUNK = IDX_W * SUBS
NBUF = 2


def _emb_body(idx_hbm, table_hbm, out_hbm, idx_v, rows_v, sems, *, n_chunks,
              per_w):
  wid = lax.axis_index("s") * NUM_CORES + lax.axis_index("c")
  base = wid * per_w

  pltpu.sync_copy(idx_hbm.at[pl.ds(base, per_w)], idx_v)

  def fire(g, b):
    for j in range(SUBS):
      pltpu.async_copy(
          table_hbm.at[idx_v.at[pl.ds(g * CHUNK + j * IDX_W, IDX_W)]],
          rows_v.at[b].at[pl.ds(j * IDX_W, IDX_W)], sems[b])

  def finish(g, b):
    pltpu.make_async_copy(out_hbm.at[pl.ds(0, CHUNK)], rows_v.at[b],
                          sems[b]).wait()
    rv = rows_v.at[b]

    @plsc.parallel_loop(0, CHUNK, 1, unroll=8)
    def _(r):
      for l in range(D_MODEL // LANES):
        sl = pl.ds(l * LANES, LANES)
        rv[r, sl] = rv[r, sl] * SCALE

    pltpu.sync_copy(rv, out_hbm.at[pl.ds(base + g * CHUNK, CHUNK)])

  fire(0, 0)

  @pl.loop(0, n_chunks, step=NBUF)
  def _(g):
    fire(g + 1, 1)
    finish(g, 0)

    @pl.when(g + 2 < n_chunks)
    def _():
      fire(g + 2, 0)

    finish(g + 1, 1)


def kernel(x, table):
  b0, b1 = x.shape
  b = b0 * b1
  assert b % (NUM_WORKERS * CHUNK * NBUF) == 0
  per_w = b // NUM_WORKERS
  n_chunks = per_w // CHUNK

  idx = x.reshape(b).astype(jnp.int32)

  mesh = plsc.VectorSubcoreMesh(core_axis_name="c", subcore_axis_name="s")
  emb = pl.kernel(
      functools.partial(_emb_body, n_chunks=n_chunks, per_w=per_w),
      out_type=jax.ShapeDtypeStruct((b, D_MODEL), jnp.float32),
      mesh=mesh,
      compiler_params=pltpu.CompilerParams(use_tc_tiling_on_sc=False),
      scratch_types=[
          pltpu.VMEM((per_w,), jnp.int32),
          pltpu.VMEM((NBUF, CHUNK, D_MODEL), jnp.float32),
          [pltpu.SemaphoreType.DMA for _ in range(NBUF)],
      ],
  )
  out = emb(idx, table)
  return out.reshape(b0, b1, D_MODEL)
